# baseline (speedup 1.0000x reference)
import functools

import jax
import jax.numpy as jnp
from jax import lax
from jax.experimental import pallas as pl
from jax.experimental.pallas import tpu as pltpu
from jax.experimental.pallas import tpu_sc as plsc

E = 8
N = 4096
D = 2048
K = 2
L = 16

NC = 2
NS = 16
NW = NC * NS
TPW = N // NW
T = 8
NSUB = TPW // T
ROWS = T

TOK_BLK = 1024


def _gating_body(h_ref, gw_ref, gb_ref, fi0_ref, fi1_ref, w0_ref):
    blk = pl.program_id(0)
    h = h_ref[...]
    gw = gw_ref[...]
    gb = gb_ref[...]
    scores = lax.dot_general(h, gw, (((1,), (1,)), ((), ())),
                             preferred_element_type=jnp.float32) + gb
    iota_e = lax.broadcasted_iota(jnp.int32, (TOK_BLK, E), 1)
    m0 = jnp.max(scores, axis=1, keepdims=True)
    a0 = jnp.min(jnp.where(scores == m0, iota_e, E), axis=1, keepdims=True)
    masked = jnp.where(iota_e == a0, -jnp.inf, scores)
    m1 = jnp.max(masked, axis=1, keepdims=True)
    w0 = 1.0 / (1.0 + jnp.exp(m1 - m0))
    w0_ref[...] = jnp.broadcast_to(w0, (TOK_BLK, L))
    tok1 = blk * TOK_BLK + lax.broadcasted_iota(jnp.int32, (TOK_BLK,), 0)
    a0_1 = jnp.min(jnp.where(scores == m0, iota_e, E), axis=1)
    a1_1 = jnp.min(jnp.where(masked == m1, iota_e, E), axis=1)
    fi0_ref[...] = a0_1 * N + tok1
    fi1_ref[...] = a1_1 * N + tok1


def _gating(hidden_states, gate_w, gate_b2d):
    return pl.pallas_call(
        _gating_body,
        grid=(N // TOK_BLK,),
        in_specs=[
            pl.BlockSpec((TOK_BLK, D), lambda i: (i, 0)),
            pl.BlockSpec((E, D), lambda i: (0, 0)),
            pl.BlockSpec((1, E), lambda i: (0, 0)),
        ],
        out_specs=[
            pl.BlockSpec((TOK_BLK,), lambda i: (i,)),
            pl.BlockSpec((TOK_BLK,), lambda i: (i,)),
            pl.BlockSpec((TOK_BLK, L), lambda i: (i, 0)),
        ],
        out_shape=[
            jax.ShapeDtypeStruct((N,), jnp.int32),
            jax.ShapeDtypeStruct((N,), jnp.int32),
            jax.ShapeDtypeStruct((N, L), jnp.float32),
        ],
    )(hidden_states, gate_w, gate_b2d)


def _combine_body(eo_ref, fi0_ref, fi1_ref, w0_ref, out_ref,
                  i0_v, i1_v, w0_v,
                  r00, r01, r10, r11, o0, o1,
                  ga0, ga1, gb0, gb1, s0, s1):
    wid = lax.axis_index("s") * NC + lax.axis_index("c")
    tok0 = wid * TPW
    pltpu.sync_copy(fi0_ref.at[pl.ds(tok0, TPW)], i0_v)
    pltpu.sync_copy(fi1_ref.at[pl.ds(tok0, TPW)], i1_v)
    pltpu.sync_copy(w0_ref.at[pl.ds(tok0, TPW), :], w0_v)

    r0bufs = (r00, r01)
    r1bufs = (r10, r11)
    obufs = (o0, o1)
    g0sems = (ga0, ga1)
    g1sems = (gb0, gb1)
    ssems = (s0, s1)

    def start_gather(j, b):
        pltpu.async_copy(
            eo_ref.at[i0_v.at[pl.ds(j * T, T)]], r0bufs[b], g0sems[b])
        pltpu.async_copy(
            eo_ref.at[i1_v.at[pl.ds(j * T, T)]], r1bufs[b], g1sems[b])

    def wait_gather(j, b):
        pltpu.make_async_copy(
            eo_ref.at[i0_v.at[pl.ds(j * T, T)]], r0bufs[b], g0sems[b]).wait()
        pltpu.make_async_copy(
            eo_ref.at[i1_v.at[pl.ds(j * T, T)]], r1bufs[b], g1sems[b]).wait()

    def start_store(j, b):
        pltpu.async_copy(obufs[b], out_ref.at[pl.ds(tok0 + j * T, T)], ssems[b])

    def wait_store(j, b):
        pltpu.make_async_copy(
            obufs[b], out_ref.at[pl.ds(tok0 + j * T, T)], ssems[b]).wait()

    start_gather(0, 0)

    def pair_body(gp, _):
        for b in (0, 1):
            j = 2 * gp + b
            if b == 0:
                start_gather(j + 1, 1)
            else:
                @pl.when(gp < NSUB // 2 - 1)
                def _():
                    start_gather(j + 1, 0)
            wait_gather(j, b)

            @pl.when(j >= 2)
            def _():
                wait_store(j, b)

            def tok_body(t, _, jj=j, bb=b):
                wv0 = w0_v[jj * T + t]

                @plsc.parallel_loop(0, D, L, unroll=8)
                def d_body(c):
                    r1 = r1bufs[bb][t, pl.ds(c, L)]
                    obufs[bb][t, pl.ds(c, L)] = (
                        r1 + (r0bufs[bb][t, pl.ds(c, L)] - r1) * wv0)

                return 0

            lax.fori_loop(0, T, tok_body, 0)
            start_store(j, b)
        return 0

    lax.fori_loop(0, NSUB // 2, pair_body, 0)
    for b in (0, 1):
        wait_store(NSUB - 2 + b, b)


@functools.cache
def _make_combine():
    return pl.kernel(
        _combine_body,
        out_type=jax.ShapeDtypeStruct((N, D), jnp.float32),
        mesh=plsc.VectorSubcoreMesh(core_axis_name="c", subcore_axis_name="s",
                                    num_cores=NC, num_subcores=NS),
        scratch_types=[
            pltpu.VMEM((TPW,), jnp.int32),
            pltpu.VMEM((TPW,), jnp.int32),
            pltpu.VMEM((TPW, L), jnp.float32),
            pltpu.VMEM((ROWS, D), jnp.float32),
            pltpu.VMEM((ROWS, D), jnp.float32),
            pltpu.VMEM((ROWS, D), jnp.float32),
            pltpu.VMEM((ROWS, D), jnp.float32),
            pltpu.VMEM((T, D), jnp.float32),
            pltpu.VMEM((T, D), jnp.float32),
            pltpu.SemaphoreType.DMA,
            pltpu.SemaphoreType.DMA,
            pltpu.SemaphoreType.DMA,
            pltpu.SemaphoreType.DMA,
            pltpu.SemaphoreType.DMA,
            pltpu.SemaphoreType.DMA,
        ],
    )


def kernel(hidden_states, expert_outputs, gate_w, gate_b):
    fi0, fi1, w0s = _gating(hidden_states, gate_w, gate_b.reshape(1, E))
    eo_flat = expert_outputs.reshape(E * N, D)
    return _make_combine()(eo_flat, fi0, fi1, w0s)


# speedup vs baseline: 1.0226x; 1.0226x over previous
import functools

import jax
import jax.numpy as jnp
from jax import lax
from jax.experimental import pallas as pl
from jax.experimental.pallas import tpu as pltpu
from jax.experimental.pallas import tpu_sc as plsc

E = 8
N = 4096
D = 2048
K = 2
L = 16

NC = 2
NS = 16
NW = NC * NS
TPW = N // NW
T = 8
NSUB = TPW // T
ROWS = T

TOK_BLK = 512


def _gating_body(h_ref, gw_ref, gb_ref, fi0_ref, fi1_ref, w0_ref):
    blk = pl.program_id(0)
    h = h_ref[...]
    gw = gw_ref[...]
    gb = gb_ref[...]
    scores = lax.dot_general(h, gw, (((1,), (1,)), ((), ())),
                             preferred_element_type=jnp.float32) + gb
    iota_e = lax.broadcasted_iota(jnp.int32, (TOK_BLK, E), 1)
    m0 = jnp.max(scores, axis=1, keepdims=True)
    a0 = jnp.min(jnp.where(scores == m0, iota_e, E), axis=1, keepdims=True)
    masked = jnp.where(iota_e == a0, -jnp.inf, scores)
    m1 = jnp.max(masked, axis=1, keepdims=True)
    w0 = 1.0 / (1.0 + jnp.exp(m1 - m0))
    w0_ref[...] = jnp.broadcast_to(w0, (TOK_BLK, L))
    tok1 = blk * TOK_BLK + lax.broadcasted_iota(jnp.int32, (TOK_BLK,), 0)
    a0_1 = jnp.min(jnp.where(scores == m0, iota_e, E), axis=1)
    a1_1 = jnp.min(jnp.where(masked == m1, iota_e, E), axis=1)
    fi0_ref[...] = a0_1 * N + tok1
    fi1_ref[...] = a1_1 * N + tok1


def _gating(hidden_states, gate_w, gate_b2d):
    return pl.pallas_call(
        _gating_body,
        grid=(N // TOK_BLK,),
        in_specs=[
            pl.BlockSpec((TOK_BLK, D), lambda i: (i, 0)),
            pl.BlockSpec((E, D), lambda i: (0, 0)),
            pl.BlockSpec((1, E), lambda i: (0, 0)),
        ],
        out_specs=[
            pl.BlockSpec((TOK_BLK,), lambda i: (i,)),
            pl.BlockSpec((TOK_BLK,), lambda i: (i,)),
            pl.BlockSpec((TOK_BLK, L), lambda i: (i, 0)),
        ],
        out_shape=[
            jax.ShapeDtypeStruct((N,), jnp.int32),
            jax.ShapeDtypeStruct((N,), jnp.int32),
            jax.ShapeDtypeStruct((N, L), jnp.float32),
        ],
    )(hidden_states, gate_w, gate_b2d)


def _combine_body(eo_ref, fi0_ref, fi1_ref, w0_ref, out_ref,
                  i0_v, i1_v, w0_v,
                  r00, r01, r10, r11, o0, o1,
                  ga0, ga1, gb0, gb1, s0, s1):
    wid = lax.axis_index("s") * NC + lax.axis_index("c")
    tok0 = wid * TPW
    pltpu.sync_copy(fi0_ref.at[pl.ds(tok0, TPW)], i0_v)
    pltpu.sync_copy(fi1_ref.at[pl.ds(tok0, TPW)], i1_v)
    pltpu.sync_copy(w0_ref.at[pl.ds(tok0, TPW), :], w0_v)

    r0bufs = (r00, r01)
    r1bufs = (r10, r11)
    obufs = (o0, o1)
    g0sems = (ga0, ga1)
    g1sems = (gb0, gb1)
    ssems = (s0, s1)

    def start_gather(j, b):
        pltpu.async_copy(
            eo_ref.at[i0_v.at[pl.ds(j * T, T)]], r0bufs[b], g0sems[b])
        pltpu.async_copy(
            eo_ref.at[i1_v.at[pl.ds(j * T, T)]], r1bufs[b], g1sems[b])

    def wait_gather(j, b):
        pltpu.make_async_copy(
            eo_ref.at[i0_v.at[pl.ds(j * T, T)]], r0bufs[b], g0sems[b]).wait()
        pltpu.make_async_copy(
            eo_ref.at[i1_v.at[pl.ds(j * T, T)]], r1bufs[b], g1sems[b]).wait()

    def start_store(j, b):
        pltpu.async_copy(obufs[b], out_ref.at[pl.ds(tok0 + j * T, T)], ssems[b])

    def wait_store(j, b):
        pltpu.make_async_copy(
            obufs[b], out_ref.at[pl.ds(tok0 + j * T, T)], ssems[b]).wait()

    start_gather(0, 0)

    def pair_body(gp, _):
        for b in (0, 1):
            j = 2 * gp + b
            if b == 0:
                start_gather(j + 1, 1)
            else:
                @pl.when(gp < NSUB // 2 - 1)
                def _():
                    start_gather(j + 1, 0)
            wait_gather(j, b)

            @pl.when(j >= 2)
            def _():
                wait_store(j, b)

            def tok_body(t, _, jj=j, bb=b):
                wv0 = w0_v[jj * T + t]

                @plsc.parallel_loop(0, D, L, unroll=8)
                def d_body(c):
                    r1 = r1bufs[bb][t, pl.ds(c, L)]
                    obufs[bb][t, pl.ds(c, L)] = (
                        r1 + (r0bufs[bb][t, pl.ds(c, L)] - r1) * wv0)

                return 0

            lax.fori_loop(0, T, tok_body, 0)
            start_store(j, b)
        return 0

    lax.fori_loop(0, NSUB // 2, pair_body, 0)
    for b in (0, 1):
        wait_store(NSUB - 2 + b, b)


@functools.cache
def _make_combine():
    return pl.kernel(
        _combine_body,
        out_type=jax.ShapeDtypeStruct((N, D), jnp.float32),
        mesh=plsc.VectorSubcoreMesh(core_axis_name="c", subcore_axis_name="s",
                                    num_cores=NC, num_subcores=NS),
        scratch_types=[
            pltpu.VMEM((TPW,), jnp.int32),
            pltpu.VMEM((TPW,), jnp.int32),
            pltpu.VMEM((TPW, L), jnp.float32),
            pltpu.VMEM((ROWS, D), jnp.float32),
            pltpu.VMEM((ROWS, D), jnp.float32),
            pltpu.VMEM((ROWS, D), jnp.float32),
            pltpu.VMEM((ROWS, D), jnp.float32),
            pltpu.VMEM((T, D), jnp.float32),
            pltpu.VMEM((T, D), jnp.float32),
            pltpu.SemaphoreType.DMA,
            pltpu.SemaphoreType.DMA,
            pltpu.SemaphoreType.DMA,
            pltpu.SemaphoreType.DMA,
            pltpu.SemaphoreType.DMA,
            pltpu.SemaphoreType.DMA,
        ],
    )


def kernel(hidden_states, expert_outputs, gate_w, gate_b):
    fi0, fi1, w0s = _gating(hidden_states, gate_w, gate_b.reshape(1, E))
    eo_flat = expert_outputs.reshape(E * N, D)
    return _make_combine()(eo_flat, fi0, fi1, w0s)
